# baseline (speedup 1.0000x reference)
import functools

import jax
import jax.numpy as jnp
from jax import lax
from jax.experimental import pallas as pl
from jax.experimental.pallas import tpu as pltpu
from jax.experimental.pallas import tpu_sc as plsc

N_C = 4096
M_F = 16384
D_IN = 512
D_SKIP = 256
D_OUT = 512
BM = 256

SC_LANES = 16
SC_NW = 32
SC_P = M_F // SC_NW
SC_CP = 16


def _knn_body(posT_ref, batchf_ref, ps_ref, bsf_ref, xs_ref,
              w2t_ref, b_ref, idx_ref, wn_ref, skip_ref):
    q0 = ps_ref[:, 0:1]
    q1 = ps_ref[:, 1:2]
    q2 = ps_ref[:, 2:3]
    p0 = posT_ref[0:1, :]
    p1 = posT_ref[1:2, :]
    p2 = posT_ref[2:3, :]
    d2 = (q0 - p0) ** 2 + (q1 - p1) ** 2 + (q2 - p2) ** 2
    same = bsf_ref[...] == batchf_ref[...]
    masked = jnp.where(same, d2, jnp.inf)

    iota = jax.lax.broadcasted_iota(jnp.int32, (1, N_C), 1).astype(jnp.float32)
    idx_cols = []
    w_cols = []
    wsum = jnp.zeros((BM, 1), jnp.float32)
    for k in range(3):
        mval = jnp.min(masked, axis=1, keepdims=True)
        tie = jnp.where(masked == mval, iota, float(N_C))
        midx = jnp.min(tie, axis=1, keepdims=True)
        w = 1.0 / (jnp.sqrt(mval) + 1e-8)
        idx_cols.append(midx)
        w_cols.append(w)
        wsum = wsum + w
        if k < 2:
            hit = iota == midx
            masked = jnp.where(hit, jnp.inf, masked)

    inv = 1.0 / (wsum + 1e-8)
    idx_ref[...] = jnp.concatenate(idx_cols, axis=1).astype(jnp.int32)
    wn_ref[...] = jnp.concatenate(w_cols, axis=1) * inv
    skip_ref[...] = (jnp.dot(xs_ref[...].astype(jnp.bfloat16), w2t_ref[...],
                             preferred_element_type=jnp.float32)
                     + b_ref[...])


def _xw_body(x_ref, w1t_ref, xw_ref):
    xw_ref[...] = jnp.dot(x_ref[...].astype(jnp.bfloat16), w1t_ref[...],
                          preferred_element_type=jnp.float32)


def _sc_body(xw_hbm, idx_hbm, wrep_hbm, skip_hbm, out_hbm,
             idx_v, rows_v, wrep_v, acc_v, sem):
    wid = lax.axis_index("s") * 2 + lax.axis_index("c")

    def chunk(g, carry):
        base = wid * SC_P + g * SC_CP
        pltpu.sync_copy(idx_hbm.at[pl.ds(3 * base, 3 * SC_CP)], idx_v)
        pltpu.sync_copy(wrep_hbm.at[pl.ds(3 * base, 3 * SC_CP)], wrep_v)
        pltpu.sync_copy(skip_hbm.at[pl.ds(base, SC_CP)], acc_v)
        pltpu.async_copy(xw_hbm.at[idx_v], rows_v, sem).wait()

        def point(p, carry2):
            w0 = wrep_v[3 * p, :]
            w1 = wrep_v[3 * p + 1, :]
            w2 = wrep_v[3 * p + 2, :]
            for v in range(D_OUT // SC_LANES):
                sl = pl.ds(SC_LANES * v, SC_LANES)
                acc_v[p, sl] = (acc_v[p, sl]
                                + w0 * rows_v[3 * p, sl]
                                + w1 * rows_v[3 * p + 1, sl]
                                + w2 * rows_v[3 * p + 2, sl])
            return carry2

        lax.fori_loop(0, SC_CP, point, 0)
        pltpu.sync_copy(acc_v, out_hbm.at[pl.ds(base, SC_CP)])
        return carry

    lax.fori_loop(0, SC_P // SC_CP, chunk, 0)


@jax.jit
def kernel(x, pos, batch, x_skip, pos_skip, batch_skip, W, b):
    posT = pos.T
    batchf = batch.astype(jnp.float32).reshape(1, N_C)
    bsf = batch_skip.astype(jnp.float32).reshape(M_F, 1)
    WT = W.T.astype(jnp.bfloat16)
    w1t = WT[:D_IN]
    w2t = WT[D_IN:]
    b2 = b.reshape(1, D_OUT)

    grid = (M_F // BM,)
    idx, wn, skip = pl.pallas_call(
        _knn_body,
        grid=grid,
        in_specs=[
            pl.BlockSpec((3, N_C), lambda i: (0, 0)),
            pl.BlockSpec((1, N_C), lambda i: (0, 0)),
            pl.BlockSpec((BM, 3), lambda i: (i, 0)),
            pl.BlockSpec((BM, 1), lambda i: (i, 0)),
            pl.BlockSpec((BM, D_SKIP), lambda i: (i, 0)),
            pl.BlockSpec((D_SKIP, D_OUT), lambda i: (0, 0)),
            pl.BlockSpec((1, D_OUT), lambda i: (0, 0)),
        ],
        out_specs=[
            pl.BlockSpec((BM, 3), lambda i: (i, 0)),
            pl.BlockSpec((BM, 3), lambda i: (i, 0)),
            pl.BlockSpec((BM, D_OUT), lambda i: (i, 0)),
        ],
        out_shape=[
            jax.ShapeDtypeStruct((M_F, 3), jnp.int32),
            jax.ShapeDtypeStruct((M_F, 3), jnp.float32),
            jax.ShapeDtypeStruct((M_F, D_OUT), jnp.float32),
        ],
    )(posT, batchf, pos_skip, bsf, x_skip, w2t, b2)

    xw = pl.pallas_call(
        _xw_body,
        out_shape=jax.ShapeDtypeStruct((N_C, D_IN), jnp.float32),
    )(x, w1t)

    idx_flat = idx.reshape(M_F * 3)
    wrep = jnp.broadcast_to(wn.reshape(M_F * 3, 1),
                            (M_F * 3, SC_LANES))

    mesh = plsc.VectorSubcoreMesh(core_axis_name="c", subcore_axis_name="s")
    sc = pl.kernel(
        _sc_body,
        out_type=jax.ShapeDtypeStruct((M_F, D_OUT), jnp.float32),
        mesh=mesh,
        scratch_types=[
            pltpu.VMEM((3 * SC_CP,), jnp.int32),
            pltpu.VMEM((3 * SC_CP, D_OUT), jnp.float32),
            pltpu.VMEM((3 * SC_CP, SC_LANES), jnp.float32),
            pltpu.VMEM((SC_CP, D_OUT), jnp.float32),
            pltpu.SemaphoreType.DMA,
        ],
    )
    y = sc(xw, idx_flat, wrep, skip)
    return y


# speedup vs baseline: 1.0926x; 1.0926x over previous
import functools

import jax
import jax.numpy as jnp
from jax import lax
from jax.experimental import pallas as pl
from jax.experimental.pallas import tpu as pltpu
from jax.experimental.pallas import tpu_sc as plsc

N_C = 4096
M_F = 16384
D_IN = 512
D_SKIP = 256
D_OUT = 512
BM = 256

SC_LANES = 16
SC_NW = 32
SC_P = M_F // SC_NW
SC_CP = 16


def _knn_body(posT_ref, batchf_ref, ps_ref, bsf_ref, xs_ref,
              w2t_ref, b_ref, idx_ref, wn_ref, skip_ref):
    q0 = ps_ref[:, 0:1]
    q1 = ps_ref[:, 1:2]
    q2 = ps_ref[:, 2:3]
    p0 = posT_ref[0:1, :]
    p1 = posT_ref[1:2, :]
    p2 = posT_ref[2:3, :]
    d2 = (q0 - p0) ** 2 + (q1 - p1) ** 2 + (q2 - p2) ** 2
    same = bsf_ref[...] == batchf_ref[...]
    masked = jnp.where(same, d2, jnp.inf)

    iota = jax.lax.broadcasted_iota(jnp.int32, (1, N_C), 1).astype(jnp.float32)
    idx_cols = []
    w_cols = []
    wsum = jnp.zeros((BM, 1), jnp.float32)
    for k in range(3):
        mval = jnp.min(masked, axis=1, keepdims=True)
        tie = jnp.where(masked == mval, iota, float(N_C))
        midx = jnp.min(tie, axis=1, keepdims=True)
        w = 1.0 / (jnp.sqrt(mval) + 1e-8)
        idx_cols.append(midx)
        w_cols.append(w)
        wsum = wsum + w
        if k < 2:
            hit = iota == midx
            masked = jnp.where(hit, jnp.inf, masked)

    inv = 1.0 / (wsum + 1e-8)
    idx_ref[...] = jnp.concatenate(idx_cols, axis=1).astype(jnp.int32)
    wn_ref[...] = jnp.concatenate(w_cols, axis=1) * inv
    skip_ref[...] = (jnp.dot(xs_ref[...].astype(jnp.bfloat16), w2t_ref[...],
                             preferred_element_type=jnp.float32)
                     + b_ref[...])


def _xw_body(x_ref, w1t_ref, xw_ref):
    xw_ref[...] = jnp.dot(x_ref[...].astype(jnp.bfloat16), w1t_ref[...],
                          preferred_element_type=jnp.float32)


def _sc_body(xw_hbm, idx_hbm, wrep_hbm, skip_hbm, out_hbm,
             idx0, idx1, rows0, rows1, wrep0, wrep1, acc0, acc1,
             sem0, sem1):
    wid = lax.axis_index("s") * 2 + lax.axis_index("c")
    nchunk = SC_P // SC_CP

    def load_aux(g, idx_v, wrep_v, acc_v):
        base = wid * SC_P + g * SC_CP
        pltpu.sync_copy(idx_hbm.at[pl.ds(3 * base, 3 * SC_CP)], idx_v)
        pltpu.sync_copy(wrep_hbm.at[pl.ds(3 * base, 3 * SC_CP)], wrep_v)
        pltpu.sync_copy(skip_hbm.at[pl.ds(base, SC_CP)], acc_v)

    def compute_store(g, rows_v, wrep_v, acc_v):
        def point(p, carry2):
            w0 = wrep_v[3 * p, :]
            w1 = wrep_v[3 * p + 1, :]
            w2 = wrep_v[3 * p + 2, :]
            for v in range(D_OUT // SC_LANES):
                sl = pl.ds(SC_LANES * v, SC_LANES)
                acc_v[p, sl] = (acc_v[p, sl]
                                + w0 * rows_v[3 * p, sl]
                                + w1 * rows_v[3 * p + 1, sl]
                                + w2 * rows_v[3 * p + 2, sl])
            return carry2

        lax.fori_loop(0, SC_CP, point, 0)
        base = wid * SC_P + g * SC_CP
        pltpu.sync_copy(acc_v, out_hbm.at[pl.ds(base, SC_CP)])

    load_aux(0, idx0, wrep0, acc0)
    gather0 = pltpu.async_copy(xw_hbm.at[idx0], rows0, sem0)
    load_aux(1, idx1, wrep1, acc1)
    gather1 = pltpu.async_copy(xw_hbm.at[idx1], rows1, sem1)

    def body(h, carry):
        g0 = 2 * h
        g1 = g0 + 1
        pltpu.make_async_copy(xw_hbm.at[idx0], rows0, sem0).wait()
        compute_store(g0, rows0, wrep0, acc0)

        @pl.when(h < nchunk // 2 - 1)
        def _():
            load_aux(g0 + 2, idx0, wrep0, acc0)
            pltpu.async_copy(xw_hbm.at[idx0], rows0, sem0)

        pltpu.make_async_copy(xw_hbm.at[idx1], rows1, sem1).wait()
        compute_store(g1, rows1, wrep1, acc1)

        @pl.when(h < nchunk // 2 - 1)
        def _():
            load_aux(g1 + 2, idx1, wrep1, acc1)
            pltpu.async_copy(xw_hbm.at[idx1], rows1, sem1)

        return carry

    lax.fori_loop(0, nchunk // 2, body, 0)


@jax.jit
def kernel(x, pos, batch, x_skip, pos_skip, batch_skip, W, b):
    posT = pos.T
    batchf = batch.astype(jnp.float32).reshape(1, N_C)
    bsf = batch_skip.astype(jnp.float32).reshape(M_F, 1)
    WT = W.T.astype(jnp.bfloat16)
    w1t = WT[:D_IN]
    w2t = WT[D_IN:]
    b2 = b.reshape(1, D_OUT)

    grid = (M_F // BM,)
    idx, wn, skip = pl.pallas_call(
        _knn_body,
        grid=grid,
        in_specs=[
            pl.BlockSpec((3, N_C), lambda i: (0, 0)),
            pl.BlockSpec((1, N_C), lambda i: (0, 0)),
            pl.BlockSpec((BM, 3), lambda i: (i, 0)),
            pl.BlockSpec((BM, 1), lambda i: (i, 0)),
            pl.BlockSpec((BM, D_SKIP), lambda i: (i, 0)),
            pl.BlockSpec((D_SKIP, D_OUT), lambda i: (0, 0)),
            pl.BlockSpec((1, D_OUT), lambda i: (0, 0)),
        ],
        out_specs=[
            pl.BlockSpec((BM, 3), lambda i: (i, 0)),
            pl.BlockSpec((BM, 3), lambda i: (i, 0)),
            pl.BlockSpec((BM, D_OUT), lambda i: (i, 0)),
        ],
        out_shape=[
            jax.ShapeDtypeStruct((M_F, 3), jnp.int32),
            jax.ShapeDtypeStruct((M_F, 3), jnp.float32),
            jax.ShapeDtypeStruct((M_F, D_OUT), jnp.float32),
        ],
    )(posT, batchf, pos_skip, bsf, x_skip, w2t, b2)

    xw = pl.pallas_call(
        _xw_body,
        out_shape=jax.ShapeDtypeStruct((N_C, D_IN), jnp.float32),
    )(x, w1t)

    idx_flat = idx.reshape(M_F * 3)
    wrep = jnp.broadcast_to(wn.reshape(M_F * 3, 1),
                            (M_F * 3, SC_LANES))

    mesh = plsc.VectorSubcoreMesh(core_axis_name="c", subcore_axis_name="s")
    sc = pl.kernel(
        _sc_body,
        out_type=jax.ShapeDtypeStruct((M_F, D_OUT), jnp.float32),
        mesh=mesh,
        scratch_types=[
            pltpu.VMEM((3 * SC_CP,), jnp.int32),
            pltpu.VMEM((3 * SC_CP,), jnp.int32),
            pltpu.VMEM((3 * SC_CP, D_OUT), jnp.float32),
            pltpu.VMEM((3 * SC_CP, D_OUT), jnp.float32),
            pltpu.VMEM((3 * SC_CP, SC_LANES), jnp.float32),
            pltpu.VMEM((3 * SC_CP, SC_LANES), jnp.float32),
            pltpu.VMEM((SC_CP, D_OUT), jnp.float32),
            pltpu.VMEM((SC_CP, D_OUT), jnp.float32),
            pltpu.SemaphoreType.DMA,
            pltpu.SemaphoreType.DMA,
        ],
    )
    y = sc(xw, idx_flat, wrep, skip)
    return y


# speedup vs baseline: 1.1730x; 1.0736x over previous
import jax
import jax.numpy as jnp
from jax import lax
from jax.experimental import pallas as pl
from jax.experimental.pallas import tpu as pltpu
from jax.experimental.pallas import tpu_sc as plsc

N_C = 4096
M_F = 16384
D_IN = 512
D_SKIP = 256
D_OUT = 512
BM = 256
N_XW = N_C // BM

SC_LANES = 16
SC_NW = 32
SC_P = M_F // SC_NW
SC_CP = 16
SC_R = 4 * SC_CP


def _tc_body(posT_ref, batchf_ref, x_ref, ps_ref, bsf_ref, xs_ref,
             w1t_ref, w2t_ref, b_ref, t_ref, idx_ref, wn_ref):
    i = pl.program_id(0)

    @pl.when(i < N_XW)
    def _xw():
        t_ref[...] = jnp.dot(x_ref[...].astype(jnp.bfloat16), w1t_ref[...],
                             preferred_element_type=jnp.float32)

    @pl.when(i >= N_XW)
    def _knn():
        q0 = ps_ref[:, 0:1]
        q1 = ps_ref[:, 1:2]
        q2 = ps_ref[:, 2:3]
        p0 = posT_ref[0:1, :]
        p1 = posT_ref[1:2, :]
        p2 = posT_ref[2:3, :]
        d2 = (q0 - p0) ** 2 + (q1 - p1) ** 2 + (q2 - p2) ** 2
        same = bsf_ref[...] == batchf_ref[...]
        masked = jnp.where(same, d2, jnp.inf)

        iota = jax.lax.broadcasted_iota(
            jnp.int32, (1, N_C), 1).astype(jnp.float32)
        idx_cols = []
        w_cols = []
        wsum = jnp.zeros((BM, 1), jnp.float32)
        for k in range(3):
            mval = jnp.min(masked, axis=1, keepdims=True)
            tie = jnp.where(masked == mval, iota, float(N_C))
            midx = jnp.min(tie, axis=1, keepdims=True)
            w = 1.0 / (jnp.sqrt(mval) + 1e-8)
            idx_cols.append(midx)
            w_cols.append(w)
            wsum = wsum + w
            if k < 2:
                hit = iota == midx
                masked = jnp.where(hit, jnp.inf, masked)

        inv = 1.0 / (wsum + 1e-8)
        row0 = N_C + (i - N_XW) * BM
        self_rows = (row0 + jax.lax.broadcasted_iota(jnp.int32, (BM, 1), 0)
                     ).astype(jnp.float32)
        idx_ref[...] = jnp.concatenate(
            idx_cols + [self_rows], axis=1).astype(jnp.int32)
        wn_ref[...] = jnp.concatenate(
            [w_cols[0] * inv, w_cols[1] * inv, w_cols[2] * inv,
             jnp.ones((BM, 1), jnp.float32)], axis=1)
        t_ref[...] = (jnp.dot(xs_ref[...].astype(jnp.bfloat16), w2t_ref[...],
                              preferred_element_type=jnp.float32)
                      + b_ref[...])


def _sc_body(t_hbm, idx_hbm, wrep_hbm, out_hbm,
             idx0, idx1, rows0, rows1, wrep0, wrep1, acc0, acc1,
             semg0, semg1, semo0, semo1, semi0, semi1, semw0, semw1):
    wid = lax.axis_index("s") * 2 + lax.axis_index("c")
    nchunk = SC_P // SC_CP
    half = nchunk // 2

    def aux_slice(g):
        return pl.ds(4 * (wid * SC_P + g * SC_CP), SC_R)

    def out_slice(g):
        return pl.ds(wid * SC_P + g * SC_CP, SC_CP)

    def compute(rows_v, wrep_v, acc_v):
        def point(p, c):
            w0 = wrep_v[4 * p, :]
            w1 = wrep_v[4 * p + 1, :]
            w2 = wrep_v[4 * p + 2, :]
            w3 = wrep_v[4 * p + 3, :]
            for v in range(D_OUT // SC_LANES):
                sl = pl.ds(SC_LANES * v, SC_LANES)
                acc_v[p, sl] = (w0 * rows_v[4 * p, sl]
                                + w1 * rows_v[4 * p + 1, sl]
                                + w2 * rows_v[4 * p + 2, sl]
                                + w3 * rows_v[4 * p + 3, sl])
            return c

        lax.fori_loop(0, SC_CP, point, 0)

    pltpu.sync_copy(idx_hbm.at[aux_slice(0)], idx0)
    pltpu.sync_copy(wrep_hbm.at[aux_slice(0)], wrep0)
    pltpu.async_copy(t_hbm.at[idx0], rows0, semg0)
    pltpu.sync_copy(idx_hbm.at[aux_slice(1)], idx1)
    pltpu.sync_copy(wrep_hbm.at[aux_slice(1)], wrep1)
    pltpu.async_copy(t_hbm.at[idx1], rows1, semg1)

    def phase(h, g, idx_v, rows_v, wrep_v, acc_v, semg, semo, semi, semw):
        @pl.when(h > 0)
        def _():
            pltpu.make_async_copy(acc_v, out_hbm.at[out_slice(g)], semo).wait()

        pltpu.make_async_copy(t_hbm.at[idx_v], rows_v, semg).wait()

        @pl.when(h < half - 1)
        def _():
            pltpu.async_copy(idx_hbm.at[aux_slice(g + 2)], idx_v, semi)

        @pl.when(h > 0)
        def _():
            pltpu.make_async_copy(wrep_hbm.at[aux_slice(g)], wrep_v, semw).wait()

        compute(rows_v, wrep_v, acc_v)
        pltpu.async_copy(acc_v, out_hbm.at[out_slice(g)], semo)

        @pl.when(h < half - 1)
        def _():
            pltpu.make_async_copy(idx_hbm.at[aux_slice(g + 2)], idx_v, semi).wait()
            pltpu.async_copy(t_hbm.at[idx_v], rows_v, semg)
            pltpu.async_copy(wrep_hbm.at[aux_slice(g + 2)], wrep_v, semw)

    def body(h, carry):
        phase(h, 2 * h, idx0, rows0, wrep0, acc0, semg0, semo0, semi0, semw0)
        phase(h, 2 * h + 1, idx1, rows1, wrep1, acc1, semg1, semo1, semi1, semw1)
        return carry

    lax.fori_loop(0, half, body, 0)
    pltpu.make_async_copy(acc0, out_hbm.at[out_slice(nchunk - 2)], semo0).wait()
    pltpu.make_async_copy(acc1, out_hbm.at[out_slice(nchunk - 1)], semo1).wait()


@jax.jit
def kernel(x, pos, batch, x_skip, pos_skip, batch_skip, W, b):
    posT = pos.T
    batchf = batch.astype(jnp.float32).reshape(1, N_C)
    bsf = batch_skip.astype(jnp.float32).reshape(M_F, 1)
    WT = W.T.astype(jnp.bfloat16)
    w1t = WT[:D_IN]
    w2t = WT[D_IN:]
    b2 = b.reshape(1, D_OUT)

    grid = (N_XW + M_F // BM,)
    fine = lambda i: (jnp.maximum(i - N_XW, 0), 0)
    t, idx, wn = pl.pallas_call(
        _tc_body,
        grid=grid,
        in_specs=[
            pl.BlockSpec((3, N_C), lambda i: (0, 0)),
            pl.BlockSpec((1, N_C), lambda i: (0, 0)),
            pl.BlockSpec((BM, D_IN), lambda i: (jnp.minimum(i, N_XW - 1), 0)),
            pl.BlockSpec((BM, 3), fine),
            pl.BlockSpec((BM, 1), fine),
            pl.BlockSpec((BM, D_SKIP), fine),
            pl.BlockSpec((D_IN, D_OUT), lambda i: (0, 0)),
            pl.BlockSpec((D_SKIP, D_OUT), lambda i: (0, 0)),
            pl.BlockSpec((1, D_OUT), lambda i: (0, 0)),
        ],
        out_specs=[
            pl.BlockSpec((BM, D_OUT), lambda i: (i, 0)),
            pl.BlockSpec((BM, 4), fine),
            pl.BlockSpec((BM, 4), fine),
        ],
        out_shape=[
            jax.ShapeDtypeStruct((N_C + M_F, D_OUT), jnp.float32),
            jax.ShapeDtypeStruct((M_F, 4), jnp.int32),
            jax.ShapeDtypeStruct((M_F, 4), jnp.float32),
        ],
    )(posT, batchf, x, pos_skip, bsf, x_skip, w1t, w2t, b2)

    idx_flat = idx.reshape(M_F * 4)
    wrep = jnp.broadcast_to(wn.reshape(M_F * 4, 1),
                            (M_F * 4, SC_LANES))

    mesh = plsc.VectorSubcoreMesh(core_axis_name="c", subcore_axis_name="s")
    sc = pl.kernel(
        _sc_body,
        out_type=jax.ShapeDtypeStruct((M_F, D_OUT), jnp.float32),
        mesh=mesh,
        scratch_types=[
            pltpu.VMEM((SC_R,), jnp.int32),
            pltpu.VMEM((SC_R,), jnp.int32),
            pltpu.VMEM((SC_R, D_OUT), jnp.float32),
            pltpu.VMEM((SC_R, D_OUT), jnp.float32),
            pltpu.VMEM((SC_R, SC_LANES), jnp.float32),
            pltpu.VMEM((SC_R, SC_LANES), jnp.float32),
            pltpu.VMEM((SC_CP, D_OUT), jnp.float32),
            pltpu.VMEM((SC_CP, D_OUT), jnp.float32),
            pltpu.SemaphoreType.DMA,
            pltpu.SemaphoreType.DMA,
            pltpu.SemaphoreType.DMA,
            pltpu.SemaphoreType.DMA,
            pltpu.SemaphoreType.DMA,
            pltpu.SemaphoreType.DMA,
            pltpu.SemaphoreType.DMA,
            pltpu.SemaphoreType.DMA,
        ],
    )
    y = sc(t, idx_flat, wrep)
    return y
